# baseline (speedup 1.0000x reference)
import functools

import jax
import jax.numpy as jnp
from jax import lax
from jax.experimental import pallas as pl
from jax.experimental.pallas import tpu as pltpu
from jax.experimental.pallas import tpu_sc as plsc

N = 10000
NP = 10240
E = 320000
D = 128
DH = D // 2
NC = 2
NS = 16
NW = NC * NS
CHUNK = 128
EP = NW * 80 * CHUNK
PAD = EP - E

ROWS_PER_TILE = NP // NS
WB = ROWS_PER_TILE // CHUNK

_MESH = plsc.VectorSubcoreMesh(core_axis_name="c", subcore_axis_name="s")


def _fill_vmem(ref, rows, cols, value):
    vv = jnp.full((16,), value, jnp.float32)
    per_row = cols // 16

    def body(i, _):
        ref[i // per_row, pl.ds((i % per_row) * 16, 16)] = vv
        return 0

    lax.fori_loop(0, rows * per_row, body, 0)


_DEG_NCH = EP // NS // CHUNK


@functools.partial(
    pl.kernel,
    out_type=jax.ShapeDtypeStruct((2, NP, D), jnp.float32),
    mesh=_MESH,
    scratch_types=[
        pltpu.VMEM_SHARED((NP, D), jnp.float32),
        pltpu.VMEM((_DEG_NCH, CHUNK), jnp.int32),
        pltpu.VMEM((CHUNK, D), jnp.float32),
    ],
)
def _sc_degrees(edge_r, out_hbm, table, idx_v, buf_v):
    c = lax.axis_index("c")
    sid = lax.axis_index("s")

    _fill_vmem(buf_v, CHUNK, D, 0.0)

    def zero_slice(k, _):
        pltpu.sync_copy(buf_v, table.at[pl.ds(sid * ROWS_PER_TILE + k * CHUNK, CHUNK)])
        return 0

    lax.fori_loop(0, WB, zero_slice, 0)

    pltpu.sync_copy(edge_r.at[c, sid], idx_v)
    _fill_vmem(buf_v, CHUNK, D, 1.0)
    plsc.subcore_barrier()

    def count(j, _):
        pltpu.sync_copy(buf_v, table.at[idx_v.at[j]], add=True)
        return 0

    lax.fori_loop(0, _DEG_NCH, count, 0)
    plsc.subcore_barrier()

    def wb(k, _):
        off = sid * ROWS_PER_TILE + k * CHUNK
        pltpu.sync_copy(table.at[pl.ds(off, CHUNK)], buf_v)
        pltpu.sync_copy(buf_v, out_hbm.at[c, pl.ds(off, CHUNK)])
        return 0

    lax.fori_loop(0, WB, wb, 0)


_GS_NCH = EP // NW // CHUNK
_PH = _GS_NCH // 2


@functools.partial(
    pl.kernel,
    out_type=jax.ShapeDtypeStruct((NC, NP, D), jnp.float32),
    mesh=_MESH,
    scratch_types=[
        pltpu.VMEM_SHARED((NP, D), jnp.float32),
        pltpu.VMEM((_PH, CHUNK), jnp.int32),
        pltpu.VMEM((_GS_NCH, CHUNK), jnp.int32),
        pltpu.VMEM((2, CHUNK, D), jnp.float32),
    ],
)
def _sc_gather_scatter(h_hbm, edge_r, out_hbm, acc, src_v, dst_v, rows_v):
    c = lax.axis_index("c")
    sid = lax.axis_index("s")
    wid = sid * NC + c
    own = sid * ROWS_PER_TILE

    _fill_vmem(rows_v.at[0], CHUNK, D, 0.0)

    def zero_slice(k, _):
        pltpu.sync_copy(rows_v.at[0], acc.at[pl.ds(own + k * CHUNK, CHUNK)])
        return 0

    lax.fori_loop(0, WB, zero_slice, 0)
    pltpu.sync_copy(edge_r.at[1, wid], dst_v)
    plsc.subcore_barrier()

    @pl.loop(0, 2)
    def phase(p):
        pltpu.sync_copy(edge_r.at[0, wid, pl.ds(p * _PH, _PH)], src_v)

        def chunk(q, _):
            pltpu.sync_copy(h_hbm.at[src_v.at[q]], rows_v.at[0])
            pltpu.sync_copy(rows_v.at[0], acc.at[dst_v.at[p * _PH + q]],
                            add=True)
            return 0

        lax.fori_loop(0, _PH, chunk, 0)

    plsc.subcore_barrier()

    def wb(k, _):
        off = own + k * CHUNK
        pltpu.sync_copy(acc.at[pl.ds(off, CHUNK)], rows_v.at[0])
        pltpu.sync_copy(rows_v.at[0], out_hbm.at[c, pl.ds(off, CHUNK)])
        return 0

    lax.fori_loop(0, WB, wb, 0)


_BR = 1000


def _norm_col(deg_ref):
    deg = deg_ref[:, 0:1]
    return jnp.where(deg > 0.0, lax.rsqrt(jnp.maximum(deg, 1.0)), 0.0)


def _join_agg(agg_ref):
    return agg_ref[0] + agg_ref[1]


def _prep1_body(feat_ref, dego_ref, w_ref, out_ref):
    h = feat_ref[...] * _norm_col(dego_ref)
    out_ref[...] = jnp.dot(h, w_ref[...], preferred_element_type=jnp.float32)


def _mid_body(agg_ref, degi_ref, dego_ref, b_ref, w_ref, out_ref):
    agg = _join_agg(agg_ref)
    x = jnp.maximum(agg * _norm_col(degi_ref) + b_ref[...], 0.0)
    h = x * _norm_col(dego_ref)
    out_ref[...] = jnp.dot(h, w_ref[...], preferred_element_type=jnp.float32)


def _final_body(agg_ref, degi_ref, b_ref, out_ref):
    out_ref[...] = _join_agg(agg_ref) * _norm_col(degi_ref) + b_ref[...]


def _row_spec(shape):
    return pl.BlockSpec(shape, lambda i: (i,) + (0,) * (len(shape) - 1))


def _fixed_spec(shape):
    return pl.BlockSpec(shape, lambda i: (0,) * len(shape))


def _agg_spec():
    return pl.BlockSpec((NC, _BR, D), lambda i: (0, i, 0))


_prep1 = pl.pallas_call(
    _prep1_body,
    grid=(N // _BR,),
    in_specs=[
        _row_spec((_BR, D)),
        _row_spec((_BR, D)),
        _fixed_spec((D, D)),
    ],
    out_specs=_row_spec((_BR, D)),
    out_shape=jax.ShapeDtypeStruct((N, D), jnp.float32),
)

_mid = pl.pallas_call(
    _mid_body,
    grid=(N // _BR,),
    in_specs=[
        _agg_spec(),
        _row_spec((_BR, D)),
        _row_spec((_BR, D)),
        _fixed_spec((1, D)),
        _fixed_spec((D, D)),
    ],
    out_specs=_row_spec((_BR, D)),
    out_shape=jax.ShapeDtypeStruct((N, D), jnp.float32),
)

_final = pl.pallas_call(
    _final_body,
    grid=(N // _BR,),
    in_specs=[
        _agg_spec(),
        _row_spec((_BR, D)),
        _fixed_spec((1, D)),
    ],
    out_specs=_row_spec((_BR, D)),
    out_shape=jax.ShapeDtypeStruct((N, D), jnp.float32),
)


def kernel(features, edge_index, W1, b1, W2, b2):
    edge = edge_index.astype(jnp.int32)
    src, dst = edge[0], edge[1]
    pad_hi = jnp.full((PAD,), NP - 1, jnp.int32)
    pad_lo = jnp.zeros((PAD,), jnp.int32)
    dst_p = jnp.concatenate([dst, pad_hi])
    edge_deg = jnp.stack([jnp.concatenate([src, pad_hi]), dst_p])
    edge_deg = edge_deg.reshape(2, NS, _DEG_NCH, CHUNK)
    edge_gs = jnp.stack([jnp.concatenate([src, pad_lo]), dst_p])
    edge_gs = edge_gs.reshape(2, NW, _GS_NCH, CHUNK)
    b1r = b1.reshape(1, D)
    b2r = b2.reshape(1, D)

    degs = _sc_degrees(edge_deg)
    dego = degs[0]
    degi = degs[1]

    h1 = _prep1(features, dego, W1)
    agg1 = _sc_gather_scatter(h1, edge_gs)
    h2 = _mid(agg1, degi, dego, b1r, W2)
    agg2 = _sc_gather_scatter(h2, edge_gs)
    return _final(agg2, degi, b2r)


# speedup vs baseline: 1.1610x; 1.1610x over previous
import functools

import jax
import jax.numpy as jnp
from jax import lax
from jax.experimental import pallas as pl
from jax.experimental.pallas import tpu as pltpu
from jax.experimental.pallas import tpu_sc as plsc

N = 10000
NP = 10240
E = 320000
D = 128
DH = D // 2
NC = 2
NS = 16
NW = NC * NS
CHUNK = 128
EP = NW * 80 * CHUNK
PAD = EP - E

ROWS_PER_TILE = NP // NS
WB = ROWS_PER_TILE // CHUNK

_MESH = plsc.VectorSubcoreMesh(core_axis_name="c", subcore_axis_name="s")


def _fill_vmem(ref, rows, cols, value):
    vv = jnp.full((16,), value, jnp.float32)
    per_row = cols // 16

    def body(i, _):
        ref[i // per_row, pl.ds((i % per_row) * 16, 16)] = vv
        return 0

    lax.fori_loop(0, rows * per_row, body, 0)


_DEG_NCH = EP // NS // CHUNK


@functools.partial(
    pl.kernel,
    out_type=jax.ShapeDtypeStruct((2, NP, D), jnp.float32),
    mesh=_MESH,
    scratch_types=[
        pltpu.VMEM_SHARED((NP, D), jnp.float32),
        pltpu.VMEM((_DEG_NCH, CHUNK), jnp.int32),
        pltpu.VMEM((CHUNK, D), jnp.float32),
    ],
)
def _sc_degrees(edge_r, out_hbm, table, idx_v, buf_v):
    c = lax.axis_index("c")
    sid = lax.axis_index("s")

    _fill_vmem(buf_v, CHUNK, D, 0.0)

    def zero_slice(k, _):
        pltpu.sync_copy(buf_v, table.at[pl.ds(sid * ROWS_PER_TILE + k * CHUNK, CHUNK)])
        return 0

    lax.fori_loop(0, WB, zero_slice, 0)

    pltpu.sync_copy(edge_r.at[c, sid], idx_v)
    _fill_vmem(buf_v, CHUNK, D, 1.0)
    plsc.subcore_barrier()

    def count(j, _):
        pltpu.sync_copy(buf_v, table.at[idx_v.at[j]], add=True)
        return 0

    lax.fori_loop(0, _DEG_NCH, count, 0)
    plsc.subcore_barrier()

    def wb(k, _):
        off = sid * ROWS_PER_TILE + k * CHUNK
        pltpu.sync_copy(table.at[pl.ds(off, CHUNK)], buf_v)
        pltpu.sync_copy(buf_v, out_hbm.at[c, pl.ds(off, CHUNK)])
        return 0

    lax.fori_loop(0, WB, wb, 0)


_GS_NCH = EP // NW // CHUNK
_PH = _GS_NCH // 2


@functools.partial(
    pl.kernel,
    out_type=jax.ShapeDtypeStruct((NC, NP, D), jnp.float32),
    mesh=_MESH,
    scratch_types=[
        pltpu.VMEM_SHARED((NP, D), jnp.float32),
        pltpu.VMEM((_PH, CHUNK), jnp.int32),
        pltpu.VMEM((_GS_NCH, CHUNK), jnp.int32),
        pltpu.VMEM((2, CHUNK, D), jnp.float32),
        pltpu.SemaphoreType.DMA,
        pltpu.SemaphoreType.DMA,
    ],
)
def _sc_gather_scatter(h_hbm, edge_r, out_hbm, acc, src_v, dst_v, rows_v,
                       sem0, sem1):
    c = lax.axis_index("c")
    sid = lax.axis_index("s")
    wid = sid * NC + c
    sems = (sem0, sem1)
    own = sid * ROWS_PER_TILE

    def _wait_gather(b):
        pltpu.make_async_copy(h_hbm.at[pl.ds(0, CHUNK)], rows_v.at[b],
                              sems[b]).wait()

    _fill_vmem(rows_v.at[0], CHUNK, D, 0.0)

    def zero_slice(k, _):
        pltpu.sync_copy(rows_v.at[0], acc.at[pl.ds(own + k * CHUNK, CHUNK)])
        return 0

    lax.fori_loop(0, WB, zero_slice, 0)
    pltpu.sync_copy(edge_r.at[1, wid], dst_v)
    plsc.subcore_barrier()

    @pl.loop(0, 2)
    def phase(p):
        pltpu.sync_copy(edge_r.at[0, wid, pl.ds(p * _PH, _PH)], src_v)
        for b in range(2):
            pltpu.async_copy(h_hbm.at[src_v.at[b]], rows_v.at[b], sems[b])

        @pl.loop(0, _PH - 2, step=2)
        def pipe(k):
            for b in range(2):
                q = k + b
                _wait_gather(b)
                pltpu.sync_copy(rows_v.at[b], acc.at[dst_v.at[p * _PH + q]],
                                add=True)
                pltpu.async_copy(h_hbm.at[src_v.at[q + 2]], rows_v.at[b],
                                 sems[b])

        for b in range(2):
            _wait_gather(b)
            pltpu.sync_copy(rows_v.at[b],
                            acc.at[dst_v.at[p * _PH + _PH - 2 + b]], add=True)

    plsc.subcore_barrier()

    def wb(k, _):
        off = own + k * CHUNK
        pltpu.sync_copy(acc.at[pl.ds(off, CHUNK)], rows_v.at[0])
        pltpu.sync_copy(rows_v.at[0], out_hbm.at[c, pl.ds(off, CHUNK)])
        return 0

    lax.fori_loop(0, WB, wb, 0)


_BR = 1000


def _norm_col(deg_ref):
    deg = deg_ref[:, 0:1]
    return jnp.where(deg > 0.0, lax.rsqrt(jnp.maximum(deg, 1.0)), 0.0)


def _join_agg(agg_ref):
    return agg_ref[0] + agg_ref[1]


def _prep1_body(feat_ref, dego_ref, w_ref, out_ref):
    h = feat_ref[...] * _norm_col(dego_ref)
    out_ref[...] = jnp.dot(h, w_ref[...], preferred_element_type=jnp.float32)


def _mid_body(agg_ref, degi_ref, dego_ref, b_ref, w_ref, out_ref):
    agg = _join_agg(agg_ref)
    x = jnp.maximum(agg * _norm_col(degi_ref) + b_ref[...], 0.0)
    h = x * _norm_col(dego_ref)
    out_ref[...] = jnp.dot(h, w_ref[...], preferred_element_type=jnp.float32)


def _final_body(agg_ref, degi_ref, b_ref, out_ref):
    out_ref[...] = _join_agg(agg_ref) * _norm_col(degi_ref) + b_ref[...]


def _row_spec(shape):
    return pl.BlockSpec(shape, lambda i: (i,) + (0,) * (len(shape) - 1))


def _fixed_spec(shape):
    return pl.BlockSpec(shape, lambda i: (0,) * len(shape))


def _agg_spec():
    return pl.BlockSpec((NC, _BR, D), lambda i: (0, i, 0))


_prep1 = pl.pallas_call(
    _prep1_body,
    grid=(N // _BR,),
    in_specs=[
        _row_spec((_BR, D)),
        _row_spec((_BR, D)),
        _fixed_spec((D, D)),
    ],
    out_specs=_row_spec((_BR, D)),
    out_shape=jax.ShapeDtypeStruct((N, D), jnp.float32),
)

_mid = pl.pallas_call(
    _mid_body,
    grid=(N // _BR,),
    in_specs=[
        _agg_spec(),
        _row_spec((_BR, D)),
        _row_spec((_BR, D)),
        _fixed_spec((1, D)),
        _fixed_spec((D, D)),
    ],
    out_specs=_row_spec((_BR, D)),
    out_shape=jax.ShapeDtypeStruct((N, D), jnp.float32),
)

_final = pl.pallas_call(
    _final_body,
    grid=(N // _BR,),
    in_specs=[
        _agg_spec(),
        _row_spec((_BR, D)),
        _fixed_spec((1, D)),
    ],
    out_specs=_row_spec((_BR, D)),
    out_shape=jax.ShapeDtypeStruct((N, D), jnp.float32),
)


def kernel(features, edge_index, W1, b1, W2, b2):
    edge = edge_index.astype(jnp.int32)
    src, dst = edge[0], edge[1]
    pad_hi = jnp.full((PAD,), NP - 1, jnp.int32)
    pad_lo = jnp.zeros((PAD,), jnp.int32)
    dst_p = jnp.concatenate([dst, pad_hi])
    edge_deg = jnp.stack([jnp.concatenate([src, pad_hi]), dst_p])
    edge_deg = edge_deg.reshape(2, NS, _DEG_NCH, CHUNK)
    edge_gs = jnp.stack([jnp.concatenate([src, pad_lo]), dst_p])
    edge_gs = edge_gs.reshape(2, NW, _GS_NCH, CHUNK)
    b1r = b1.reshape(1, D)
    b2r = b2.reshape(1, D)

    degs = _sc_degrees(edge_deg)
    dego = degs[0]
    degi = degs[1]

    h1 = _prep1(features, dego, W1)
    agg1 = _sc_gather_scatter(h1, edge_gs)
    h2 = _mid(agg1, degi, dego, b1r, W2)
    agg2 = _sc_gather_scatter(h2, edge_gs)
    return _final(agg2, degi, b2r)


# speedup vs baseline: 1.1640x; 1.0026x over previous
import functools

import jax
import jax.numpy as jnp
from jax import lax
from jax.experimental import pallas as pl
from jax.experimental.pallas import tpu as pltpu
from jax.experimental.pallas import tpu_sc as plsc

N = 10000
NP = 10240
E = 320000
D = 128
DH = D // 2
NC = 2
NS = 16
NW = NC * NS
CHUNK = 128
EP = NW * 80 * CHUNK
PAD = EP - E

ROWS_PER_TILE = NP // NS
WB = ROWS_PER_TILE // CHUNK

_MESH = plsc.VectorSubcoreMesh(core_axis_name="c", subcore_axis_name="s")


def _fill_vmem(ref, rows, cols, value):
    vv = jnp.full((16,), value, jnp.float32)
    per_row = cols // 16

    def body(i, _):
        ref[i // per_row, pl.ds((i % per_row) * 16, 16)] = vv
        return 0

    lax.fori_loop(0, rows * per_row, body, 0)


_DEG_NCH = EP // NS // CHUNK
DW = D


@functools.partial(
    pl.kernel,
    out_type=jax.ShapeDtypeStruct((2, NP, DW), jnp.float32),
    mesh=_MESH,
    scratch_types=[
        pltpu.VMEM_SHARED((NP, DW), jnp.float32),
        pltpu.VMEM((_DEG_NCH, CHUNK), jnp.int32),
        pltpu.VMEM((CHUNK, DW), jnp.float32),
    ],
)
def _sc_degrees(edge_r, out_hbm, table, idx_v, buf_v):
    c = lax.axis_index("c")
    sid = lax.axis_index("s")

    _fill_vmem(buf_v, CHUNK, DW, 0.0)

    def zero_slice(k, _):
        pltpu.sync_copy(buf_v, table.at[pl.ds(sid * ROWS_PER_TILE + k * CHUNK, CHUNK)])
        return 0

    lax.fori_loop(0, WB, zero_slice, 0)

    pltpu.sync_copy(edge_r.at[c, sid], idx_v)
    _fill_vmem(buf_v, CHUNK, DW, 1.0)
    plsc.subcore_barrier()

    def count(j, _):
        pltpu.sync_copy(buf_v, table.at[idx_v.at[j]], add=True)
        return 0

    lax.fori_loop(0, _DEG_NCH, count, 0)
    plsc.subcore_barrier()

    def wb(k, _):
        off = sid * ROWS_PER_TILE + k * CHUNK
        pltpu.sync_copy(table.at[pl.ds(off, CHUNK)], buf_v)
        pltpu.sync_copy(buf_v, out_hbm.at[c, pl.ds(off, CHUNK)])
        return 0

    lax.fori_loop(0, WB, wb, 0)


_GS_NCH = EP // NW // CHUNK
_PH = _GS_NCH // 2


@functools.partial(
    pl.kernel,
    out_type=jax.ShapeDtypeStruct((NC, NP, D), jnp.float32),
    mesh=_MESH,
    scratch_types=[
        pltpu.VMEM_SHARED((NP, D), jnp.float32),
        pltpu.VMEM((_PH, CHUNK), jnp.int32),
        pltpu.VMEM((_GS_NCH, CHUNK), jnp.int32),
        pltpu.VMEM((2, CHUNK, D), jnp.float32),
        pltpu.SemaphoreType.DMA,
        pltpu.SemaphoreType.DMA,
    ],
)
def _sc_gather_scatter(h_hbm, edge_r, out_hbm, acc, src_v, dst_v, rows_v,
                       sem0, sem1):
    c = lax.axis_index("c")
    sid = lax.axis_index("s")
    wid = sid * NC + c
    sems = (sem0, sem1)
    own = sid * ROWS_PER_TILE

    def _wait_gather(b):
        pltpu.make_async_copy(h_hbm.at[pl.ds(0, CHUNK)], rows_v.at[b],
                              sems[b]).wait()

    _fill_vmem(rows_v.at[0], CHUNK, D, 0.0)

    def zero_slice(k, _):
        pltpu.sync_copy(rows_v.at[0], acc.at[pl.ds(own + k * CHUNK, CHUNK)])
        return 0

    lax.fori_loop(0, WB, zero_slice, 0)
    pltpu.sync_copy(edge_r.at[1, wid], dst_v)
    plsc.subcore_barrier()

    @pl.loop(0, 2)
    def phase(p):
        pltpu.sync_copy(edge_r.at[0, wid, pl.ds(p * _PH, _PH)], src_v)
        for b in range(2):
            pltpu.async_copy(h_hbm.at[src_v.at[b]], rows_v.at[b], sems[b])

        @pl.loop(0, _PH - 2, step=2)
        def pipe(k):
            for b in range(2):
                q = k + b
                _wait_gather(b)
                pltpu.sync_copy(rows_v.at[b], acc.at[dst_v.at[p * _PH + q]],
                                add=True)
                pltpu.async_copy(h_hbm.at[src_v.at[q + 2]], rows_v.at[b],
                                 sems[b])

        for b in range(2):
            _wait_gather(b)
            pltpu.sync_copy(rows_v.at[b],
                            acc.at[dst_v.at[p * _PH + _PH - 2 + b]], add=True)

    plsc.subcore_barrier()

    def wb(k, _):
        off = own + k * CHUNK
        pltpu.sync_copy(acc.at[pl.ds(off, CHUNK)], rows_v.at[0])
        pltpu.sync_copy(rows_v.at[0], out_hbm.at[c, pl.ds(off, CHUNK)])
        return 0

    lax.fori_loop(0, WB, wb, 0)


_BR = 1000


def _norm_col(deg_ref):
    deg = deg_ref[:, 0:1]
    return jnp.where(deg > 0.0, lax.rsqrt(jnp.maximum(deg, 1.0)), 0.0)


def _join_agg(agg_ref):
    return agg_ref[0] + agg_ref[1]


def _prep1_body(feat_ref, dego_ref, w_ref, out_ref):
    h = feat_ref[...] * _norm_col(dego_ref)
    out_ref[...] = jnp.dot(h, w_ref[...], preferred_element_type=jnp.float32)


def _mid_body(agg_ref, degi_ref, dego_ref, b_ref, w_ref, out_ref):
    agg = _join_agg(agg_ref)
    x = jnp.maximum(agg * _norm_col(degi_ref) + b_ref[...], 0.0)
    h = x * _norm_col(dego_ref)
    out_ref[...] = jnp.dot(h, w_ref[...], preferred_element_type=jnp.float32)


def _final_body(agg_ref, degi_ref, b_ref, out_ref):
    out_ref[...] = _join_agg(agg_ref) * _norm_col(degi_ref) + b_ref[...]


def _row_spec(shape):
    return pl.BlockSpec(shape, lambda i: (i,) + (0,) * (len(shape) - 1))


def _fixed_spec(shape):
    return pl.BlockSpec(shape, lambda i: (0,) * len(shape))


def _agg_spec():
    return pl.BlockSpec((NC, _BR, D), lambda i: (0, i, 0))


_prep1 = pl.pallas_call(
    _prep1_body,
    grid=(N // _BR,),
    in_specs=[
        _row_spec((_BR, D)),
        _row_spec((_BR, DW)),
        _fixed_spec((D, D)),
    ],
    out_specs=_row_spec((_BR, D)),
    out_shape=jax.ShapeDtypeStruct((N, D), jnp.float32),
)

_mid = pl.pallas_call(
    _mid_body,
    grid=(N // _BR,),
    in_specs=[
        _agg_spec(),
        _row_spec((_BR, DW)),
        _row_spec((_BR, DW)),
        _fixed_spec((1, D)),
        _fixed_spec((D, D)),
    ],
    out_specs=_row_spec((_BR, D)),
    out_shape=jax.ShapeDtypeStruct((N, D), jnp.float32),
)

_final = pl.pallas_call(
    _final_body,
    grid=(N // _BR,),
    in_specs=[
        _agg_spec(),
        _row_spec((_BR, DW)),
        _fixed_spec((1, D)),
    ],
    out_specs=_row_spec((_BR, D)),
    out_shape=jax.ShapeDtypeStruct((N, D), jnp.float32),
)


def kernel(features, edge_index, W1, b1, W2, b2):
    edge = edge_index.astype(jnp.int32)
    src, dst = edge[0], edge[1]
    pad_hi = jnp.full((PAD,), NP - 1, jnp.int32)
    pad_lo = jnp.zeros((PAD,), jnp.int32)
    dst_p = jnp.concatenate([dst, pad_hi])
    edge_deg = jnp.stack([jnp.concatenate([src, pad_hi]), dst_p])
    edge_deg = edge_deg.reshape(2, NS, _DEG_NCH, CHUNK)
    edge_gs = jnp.stack([jnp.concatenate([src, pad_lo]), dst_p])
    edge_gs = edge_gs.reshape(2, NW, _GS_NCH, CHUNK)
    b1r = b1.reshape(1, D)
    b2r = b2.reshape(1, D)

    degs = _sc_degrees(edge_deg)
    dego = degs[0]
    degi = degs[1]

    h1 = _prep1(features, dego, W1)
    agg1 = _sc_gather_scatter(h1, edge_gs)
    h2 = _mid(agg1, degi, dego, b1r, W2)
    agg2 = _sc_gather_scatter(h2, edge_gs)
    return _final(agg2, degi, b2r)


# speedup vs baseline: 1.2454x; 1.0699x over previous
import functools

import jax
import jax.numpy as jnp
from jax import lax
from jax.experimental import pallas as pl
from jax.experimental.pallas import tpu as pltpu
from jax.experimental.pallas import tpu_sc as plsc

N = 10000
NP = 10240
E = 320000
D = 128
DH = D // 2
NC = 2
NS = 16
NW = NC * NS
CHUNK = 128
EP = NW * 80 * CHUNK
PAD = EP - E

ROWS_PER_TILE = NP // NS
WB = ROWS_PER_TILE // CHUNK

_MESH = plsc.VectorSubcoreMesh(core_axis_name="c", subcore_axis_name="s")


def _fill_vmem(ref, rows, cols, value):
    vv = jnp.full((16,), value, jnp.float32)
    per_row = cols // 16

    def body(i, _):
        ref[i // per_row, pl.ds((i % per_row) * 16, 16)] = vv
        return 0

    lax.fori_loop(0, rows * per_row, body, 0)


_DEG_NCH = EP // NS // CHUNK
DW = D


@functools.partial(
    pl.kernel,
    out_type=jax.ShapeDtypeStruct((2, NP, DW), jnp.float32),
    mesh=_MESH,
    scratch_types=[
        pltpu.VMEM_SHARED((NP, DW), jnp.float32),
        pltpu.VMEM((_DEG_NCH, CHUNK), jnp.int32),
        pltpu.VMEM((CHUNK, DW), jnp.float32),
    ],
)
def _sc_degrees(edge_r, out_hbm, table, idx_v, buf_v):
    c = lax.axis_index("c")
    sid = lax.axis_index("s")

    _fill_vmem(buf_v, CHUNK, DW, 0.0)

    def zero_slice(k, _):
        pltpu.sync_copy(buf_v, table.at[pl.ds(sid * ROWS_PER_TILE + k * CHUNK, CHUNK)])
        return 0

    lax.fori_loop(0, WB, zero_slice, 0)

    pltpu.sync_copy(edge_r.at[c, sid], idx_v)
    _fill_vmem(buf_v, CHUNK, DW, 1.0)
    plsc.subcore_barrier()

    def count(j, _):
        pltpu.sync_copy(buf_v, table.at[idx_v.at[j]], add=True)
        return 0

    lax.fori_loop(0, _DEG_NCH, count, 0)
    plsc.subcore_barrier()

    def wb(k, _):
        off = sid * ROWS_PER_TILE + k * CHUNK
        pltpu.sync_copy(table.at[pl.ds(off, CHUNK)], buf_v)
        pltpu.sync_copy(buf_v, out_hbm.at[c, pl.ds(off, CHUNK)])
        return 0

    lax.fori_loop(0, WB, wb, 0)


_GS_NCH = EP // NW // CHUNK
_PH = _GS_NCH // 2


@functools.partial(
    pl.kernel,
    out_type=jax.ShapeDtypeStruct((NC, NP, D), jnp.float32),
    mesh=_MESH,
    scratch_types=[
        pltpu.VMEM_SHARED((NP, D), jnp.float32),
        pltpu.VMEM((_PH, CHUNK), jnp.int32),
        pltpu.VMEM((_GS_NCH, CHUNK), jnp.int32),
        pltpu.VMEM((2, CHUNK, D), jnp.float32),
        pltpu.SemaphoreType.DMA,
        pltpu.SemaphoreType.DMA,
    ],
)
def _sc_gather_scatter(h_hbm, edge_r, out_hbm, acc, src_v, dst_v, rows_v,
                       sem0, sem1):
    c = lax.axis_index("c")
    sid = lax.axis_index("s")
    wid = sid * NC + c
    sems = (sem0, sem1)
    own = sid * ROWS_PER_TILE

    def _wait_gather(b):
        pltpu.make_async_copy(h_hbm.at[pl.ds(0, CHUNK)], rows_v.at[b],
                              sems[b]).wait()

    _fill_vmem(rows_v.at[0], CHUNK, D, 0.0)

    def zero_slice(k, _):
        pltpu.sync_copy(rows_v.at[0], acc.at[pl.ds(own + k * CHUNK, CHUNK)])
        return 0

    lax.fori_loop(0, WB, zero_slice, 0)
    pltpu.sync_copy(edge_r.at[1, wid], dst_v)
    plsc.subcore_barrier()

    @pl.loop(0, 2)
    def phase(p):
        pltpu.sync_copy(edge_r.at[0, wid, pl.ds(p * _PH, _PH)], src_v)
        for b in range(2):
            pltpu.async_copy(h_hbm.at[src_v.at[b]], rows_v.at[b], sems[b])

        @pl.loop(0, _PH - 2, step=2)
        def pipe(k):
            for b in range(2):
                q = k + b
                _wait_gather(b)
                pltpu.sync_copy(rows_v.at[b], acc.at[dst_v.at[p * _PH + q]],
                                add=True)
                pltpu.async_copy(h_hbm.at[src_v.at[q + 2]], rows_v.at[b],
                                 sems[b])

        for b in range(2):
            _wait_gather(b)
            pltpu.sync_copy(rows_v.at[b],
                            acc.at[dst_v.at[p * _PH + _PH - 2 + b]], add=True)

    plsc.subcore_barrier()

    def wb(k, _):
        off = own + k * CHUNK
        pltpu.sync_copy(acc.at[pl.ds(off, CHUNK)], rows_v.at[0])
        pltpu.sync_copy(rows_v.at[0], out_hbm.at[c, pl.ds(off, CHUNK)])
        return 0

    lax.fori_loop(0, WB, wb, 0)


_BR = 1000


def _norm_col(deg_ref):
    deg = deg_ref[:, 0:1]
    return jnp.where(deg > 0.0, lax.rsqrt(jnp.maximum(deg, 1.0)), 0.0)


def _join_agg(agg_ref):
    return agg_ref[0] + agg_ref[1]


def _prep1_body(feat_ref, dego_ref, w_ref, out_ref):
    h = feat_ref[...] * _norm_col(dego_ref)
    out_ref[...] = jnp.dot(h, w_ref[...], preferred_element_type=jnp.float32)


def _mid_body(agg_ref, degi_ref, dego_ref, b_ref, w_ref, out_ref):
    agg = _join_agg(agg_ref)
    x = jnp.maximum(agg * _norm_col(degi_ref) + b_ref[...], 0.0)
    h = x * _norm_col(dego_ref)
    out_ref[...] = jnp.dot(h, w_ref[...], preferred_element_type=jnp.float32)


def _final_body(agg_ref, degi_ref, b_ref, out_ref):
    out_ref[...] = _join_agg(agg_ref) * _norm_col(degi_ref) + b_ref[...]


def _row_spec(shape):
    return pl.BlockSpec(shape, lambda i: (i,) + (0,) * (len(shape) - 1))


def _fixed_spec(shape):
    return pl.BlockSpec(shape, lambda i: (0,) * len(shape))


def _agg_spec():
    return pl.BlockSpec((NC, _BR, D), lambda i: (0, i, 0))


_prep1 = pl.pallas_call(
    _prep1_body,
    grid=(N // _BR,),
    in_specs=[
        _row_spec((_BR, D)),
        _row_spec((_BR, DW)),
        _fixed_spec((D, D)),
    ],
    out_specs=_row_spec((_BR, D)),
    out_shape=jax.ShapeDtypeStruct((N, D), jnp.float32),
)

_mid = pl.pallas_call(
    _mid_body,
    grid=(N // _BR,),
    in_specs=[
        _agg_spec(),
        _row_spec((_BR, DW)),
        _row_spec((_BR, DW)),
        _fixed_spec((1, D)),
        _fixed_spec((D, D)),
    ],
    out_specs=_row_spec((_BR, D)),
    out_shape=jax.ShapeDtypeStruct((N, D), jnp.float32),
)

_final = pl.pallas_call(
    _final_body,
    grid=(N // _BR,),
    in_specs=[
        _agg_spec(),
        _row_spec((_BR, DW)),
        _fixed_spec((1, D)),
    ],
    out_specs=_row_spec((_BR, D)),
    out_shape=jax.ShapeDtypeStruct((N, D), jnp.float32),
)


def kernel(features, edge_index, W1, b1, W2, b2):
    edge = edge_index.astype(jnp.int32)
    src, dst = edge[0], edge[1]
    pad_hi = jnp.full((PAD,), NP - 1, jnp.int32)
    pad_lo = jnp.zeros((PAD,), jnp.int32)
    dst_p = jnp.concatenate([dst, pad_hi])
    edge_deg = jnp.stack([jnp.concatenate([src, pad_hi]), dst_p])
    edge_deg = edge_deg.reshape(2, NS, _DEG_NCH, CHUNK)
    srcw = jnp.concatenate([src, pad_lo]).reshape(NW, _GS_NCH, CHUNK)
    woff = (jnp.arange(NW, dtype=jnp.int32) % NC) * N
    edge_gs = jnp.stack([srcw + woff[:, None, None],
                         dst_p.reshape(NW, _GS_NCH, CHUNK)])
    b1r = b1.reshape(1, D)
    b2r = b2.reshape(1, D)

    degs = _sc_degrees(edge_deg)
    dego = degs[0]
    degi = degs[1]

    h1 = _prep1(features, dego, W1)
    agg1 = _sc_gather_scatter(jnp.concatenate([h1, h1]), edge_gs)
    h2 = _mid(agg1, degi, dego, b1r, W2)
    agg2 = _sc_gather_scatter(jnp.concatenate([h2, h2]), edge_gs)
    return _final(agg2, degi, b2r)
